# baseline (speedup 1.0000x reference)
import jax
import jax.numpy as jnp
from jax.experimental import pallas as pl
from jax.experimental.pallas import tpu as pltpu

_ROWS = 16384 * 26
_BLK = 2048


def _slice_body(in_ref, out_ref):
    x = in_ref[...]
    idx = jax.lax.broadcasted_iota(jnp.int32, (x.shape[0], x.shape[1] // 2), 1) * 2
    out_ref[...] = jnp.take_along_axis(x, idx, axis=1)


def kernel(inputs):
    b, s, d = inputs.shape
    x = inputs.reshape(b * s, d)
    out = pl.pallas_call(
        _slice_body,
        grid=(x.shape[0] // _BLK,),
        in_specs=[pl.BlockSpec((_BLK, d), lambda i: (i, 0))],
        out_specs=pl.BlockSpec((_BLK, d // 2), lambda i: (i, 0)),
        out_shape=jax.ShapeDtypeStruct((x.shape[0], d // 2), jnp.float32),
    )(x)
    return out.reshape(b, s, d // 2)
